# Initial kernel scaffold; baseline (speedup 1.0000x reference)
import jax
import jax.numpy as jnp
from jax.experimental import pallas as pl


def kernel(node_features, edge_index, edge_attr, W, b):
    raise NotImplementedError("write your pallas kernel here")


# speedup vs baseline: 1.1040x; 1.1040x over previous
import functools

import jax
import jax.numpy as jnp
from jax import lax
from jax.experimental import pallas as pl
from jax.experimental.pallas import tpu as pltpu
from jax.experimental.pallas import tpu_sc as plsc

N_NODES = 10000
N_EDGES = 160000
NODE_DIM = 128
EDGE_DIM = 4
HIDDEN_DIM = 256

NC = 2
NS = 16
B = 40
EA = 16
EDGES_PER_TILE = N_EDGES // NS
NCHUNKS = EDGES_PER_TILE // B
NPAD = 10240
NODES_PER_TILE = NPAD // NS
NFIN = NODES_PER_TILE // B
HALF = HIDDEN_DIM // NC
FB = HALF // 16
PROJ = EDGE_DIM * HALF


def _proj_kernel(nf_ref, a_ref, out_ref):
    out_ref[0] = jnp.dot(nf_ref[...], a_ref[0],
                         preferred_element_type=jnp.float32)


def _node_projections(node_features, a2):
    bn = 1000
    return pl.pallas_call(
        _proj_kernel,
        grid=(NC, N_NODES // bn),
        in_specs=[
            pl.BlockSpec((bn, NODE_DIM), lambda c, n: (n, 0)),
            pl.BlockSpec((1, NODE_DIM, PROJ), lambda c, n: (c, 0, 0)),
        ],
        out_specs=pl.BlockSpec((1, bn, PROJ), lambda c, n: (c, n, 0)),
        out_shape=jax.ShapeDtypeStruct((NC, NPAD, PROJ), jnp.float32),
    )(node_features, a2)


def _edge_kernel(p_hbm, srcb_hbm, dst_hbm, ea_hbm, b_hbm,
                 acc_hbm, cnt_hbm,
                 idx_v, dst_v, dst48_v, ea_v, rows_v, msg_v, cnt_l,
                 b_v, bh_v, sem, acc_sh):
    c = lax.axis_index("c")
    s = lax.axis_index("s")
    iota = lax.iota(jnp.int32, 16)

    pltpu.sync_copy(b_hbm, b_v)
    for blk in range(FB):
        o = blk * 16
        bh_v[pl.ds(o, 16)] = b_v[pl.ds(c * HALF + o, 16)]

    def zmsg(i, _):
        for blk in range(FB):
            msg_v[i, pl.ds(blk * 16, 16)] = jnp.zeros((16,), jnp.float32)
        return 0

    lax.fori_loop(0, B, zmsg, 0)

    def zcnt(i, _):
        cnt_l[pl.ds(i * 16, 16)] = jnp.zeros((16,), jnp.float32)
        return 0

    lax.fori_loop(0, NPAD // 16, zcnt, 0)

    for k in range(NFIN):
        nb = s * NODES_PER_TILE + k * B
        pltpu.sync_copy(msg_v, acc_sh.at[pl.ds(nb, B)])
    plsc.subcore_barrier()

    def chunk_body(k, _):
        base = s * EDGES_PER_TILE + k * B
        pltpu.sync_copy(srcb_hbm.at[pl.ds(c * N_EDGES + base, B)], idx_v)
        pltpu.sync_copy(dst_hbm.at[pl.ds(base, B)], dst_v)
        pltpu.sync_copy(dst_hbm.at[pl.ds(base, B)], dst48_v.at[pl.ds(0, B)])
        pltpu.sync_copy(ea_hbm.at[pl.ds(base * EA, B * EA)], ea_v)
        pltpu.async_copy(p_hbm.at[idx_v], rows_v, sem).wait()

        def edge_body(i, _):
            ev = ea_v[pl.ds(i * EA, 16)]
            e0 = ev[0]
            e1 = ev[1]
            e2 = ev[2]
            e3 = ev[3]
            for blk in range(FB):
                o = blk * 16
                v = bh_v[pl.ds(o, 16)]
                v = v + e0 * rows_v[i, pl.ds(o, 16)]
                v = v + e1 * rows_v[i, pl.ds(HALF + o, 16)]
                v = v + e2 * rows_v[i, pl.ds(2 * HALF + o, 16)]
                v = v + e3 * rows_v[i, pl.ds(3 * HALF + o, 16)]
                msg_v[i, pl.ds(o, 16)] = jnp.maximum(v, 0.0)
            return 0

        lax.fori_loop(0, B, edge_body, 0)
        pltpu.sync_copy(msg_v, acc_sh.at[dst_v], add=True)

        for g, nl in ((0, 16), (16, 16), (32, 8)):
            dv = dst48_v[pl.ds(g, 16)]
            for l in range(nl):
                d = dv[l]
                db = (d >> 4) << 4
                oneh = jnp.where(iota == (d & 15), 1.0, 0.0)
                cnt_l[pl.ds(db, 16)] = cnt_l[pl.ds(db, 16)] + oneh
        return 0

    lax.fori_loop(0, NCHUNKS, chunk_body, 0)
    plsc.subcore_barrier()

    for k in range(NFIN):
        nb = s * NODES_PER_TILE + k * B
        pltpu.sync_copy(acc_sh.at[pl.ds(nb, B)], msg_v)
        pltpu.sync_copy(msg_v, acc_hbm.at[pl.ds(c * NPAD + nb, B)])
    pltpu.sync_copy(cnt_l, cnt_hbm.at[pl.ds((c * NS + s) * NPAD, NPAD)])


@functools.partial(
    pl.kernel,
    out_type=[
        jax.ShapeDtypeStruct((NC * NPAD, HALF), jnp.float32),
        jax.ShapeDtypeStruct((NC * NS * NPAD,), jnp.float32),
    ],
    mesh=plsc.VectorSubcoreMesh(core_axis_name="c", subcore_axis_name="s"),
    scratch_types=[
        pltpu.VMEM((B,), jnp.int32),
        pltpu.VMEM((B,), jnp.int32),
        pltpu.VMEM((48,), jnp.int32),
        pltpu.VMEM((B * EA,), jnp.float32),
        pltpu.VMEM((B, PROJ), jnp.float32),
        pltpu.VMEM((B, HALF), jnp.float32),
        pltpu.VMEM((NPAD,), jnp.float32),
        pltpu.VMEM((HIDDEN_DIM,), jnp.float32),
        pltpu.VMEM((HALF,), jnp.float32),
        pltpu.SemaphoreType.DMA,
        pltpu.VMEM_SHARED((NPAD, HALF), jnp.float32),
    ],
)
def _edge_scatter(p_hbm, srcb_hbm, dst_hbm, ea_hbm, b_hbm, acc_hbm, cnt_hbm,
                  idx_v, dst_v, dst48_v, ea_v, rows_v, msg_v, cnt_l,
                  b_v, bh_v, sem, acc_sh):
    _edge_kernel(p_hbm, srcb_hbm, dst_hbm, ea_hbm, b_hbm, acc_hbm, cnt_hbm,
                 idx_v, dst_v, dst48_v, ea_v, rows_v, msg_v, cnt_l,
                 b_v, bh_v, sem, acc_sh)


def _mean_kernel(a0_ref, a1_ref, cnt_ref, out_ref):
    tot = jnp.sum(cnt_ref[:NS], axis=0)
    scale = 1.0 / jnp.maximum(tot, 1.0)
    out_ref[:, :HALF] = jnp.maximum(a0_ref[0], 0.0) * scale[:, None]
    out_ref[:, HALF:] = jnp.maximum(a1_ref[0], 0.0) * scale[:, None]


def _mean_relu(acc3, cnt3):
    bn = 2048
    return pl.pallas_call(
        _mean_kernel,
        grid=(NPAD // bn,),
        in_specs=[
            pl.BlockSpec((1, bn, HALF), lambda n: (0, n, 0)),
            pl.BlockSpec((1, bn, HALF), lambda n: (1, n, 0)),
            pl.BlockSpec((NC * NS, bn), lambda n: (0, n)),
        ],
        out_specs=pl.BlockSpec((bn, HIDDEN_DIM), lambda n: (n, 0)),
        out_shape=jax.ShapeDtypeStruct((NPAD, HIDDEN_DIM), jnp.float32),
    )(acc3, acc3, cnt3)


def kernel(node_features, edge_index, edge_attr, W, b):
    a2 = W.reshape(NC, HALF, EDGE_DIM, NODE_DIM).transpose(0, 3, 2, 1)
    a2 = a2.reshape(NC, NODE_DIM, PROJ)
    src = edge_index[0].astype(jnp.int32)
    dst = edge_index[1].astype(jnp.int32)
    srcb = jnp.concatenate([src, src + NPAD])
    ea16 = jnp.pad(edge_attr, ((0, 0), (0, EA - EDGE_DIM))).reshape(-1)

    p = _node_projections(node_features, a2)
    p_flat = p.reshape(NC * NPAD, PROJ)
    acc, cnt = _edge_scatter(p_flat, srcb, dst, ea16, b)
    acc3 = acc.reshape(NC, NPAD, HALF)
    cnt3 = cnt.reshape(NC * NS, NPAD)
    return _mean_relu(acc3, cnt3)[:N_NODES]


# speedup vs baseline: 1.3606x; 1.2325x over previous
import functools

import jax
import jax.numpy as jnp
from jax import lax
from jax.experimental import pallas as pl
from jax.experimental.pallas import tpu as pltpu
from jax.experimental.pallas import tpu_sc as plsc

N_NODES = 10000
N_EDGES = 160000
NODE_DIM = 128
EDGE_DIM = 4
HIDDEN_DIM = 256

NC = 2
NS = 16
B = 40
EA = 16
EDGES_PER_TILE = N_EDGES // NS
NCHUNKS = EDGES_PER_TILE // B
NPAD = 10240
NODES_PER_TILE = NPAD // NS
NFIN = NODES_PER_TILE // B
HALF = HIDDEN_DIM // NC
FB = HALF // 16
PROJ = EDGE_DIM * HALF
G = 10
BT = G * B
NBATCH = EDGES_PER_TILE // BT


def _proj_kernel(nf_ref, a_ref, out_ref):
    out_ref[0] = jnp.dot(nf_ref[...], a_ref[0],
                         preferred_element_type=jnp.float32)


def _node_projections(node_features, a2):
    bn = 1000
    return pl.pallas_call(
        _proj_kernel,
        grid=(NC, N_NODES // bn),
        in_specs=[
            pl.BlockSpec((bn, NODE_DIM), lambda c, n: (n, 0)),
            pl.BlockSpec((1, NODE_DIM, PROJ), lambda c, n: (c, 0, 0)),
        ],
        out_specs=pl.BlockSpec((1, bn, PROJ), lambda c, n: (c, n, 0)),
        out_shape=jax.ShapeDtypeStruct((NC, NPAD, PROJ), jnp.float32),
    )(node_features, a2)


def _edge_kernel(p_hbm, srcb_hbm, dst2_hbm, dst_hbm, ea_hbm, b_hbm,
                 acc_hbm, cnt_hbm,
                 idx_v, dst_v, dstf_v, ea_v, rows_v, msg_v, cnt_l,
                 b_v, bh_v, sem, acc_sh):
    c = lax.axis_index("c")
    s = lax.axis_index("s")
    iota = lax.iota(jnp.int32, 16)

    pltpu.sync_copy(b_hbm, b_v)
    for blk in range(FB):
        o = blk * 16
        bh_v[pl.ds(o, 16)] = b_v[pl.ds(c * HALF + o, 16)]

    def zmsg(i, _):
        for blk in range(FB):
            msg_v[i, pl.ds(blk * 16, 16)] = jnp.zeros((16,), jnp.float32)
        return 0

    lax.fori_loop(0, B, zmsg, 0)

    def zcnt(i, _):
        cnt_l[pl.ds(i * 16, 16)] = jnp.zeros((16,), jnp.float32)
        return 0

    lax.fori_loop(0, NPAD // 16, zcnt, 0)

    for k in range(NFIN):
        nb = s * NODES_PER_TILE + k * B
        pltpu.sync_copy(msg_v, acc_sh.at[pl.ds(nb, B)])
    plsc.subcore_barrier()

    bvecs = tuple(bh_v[pl.ds(blk * 16, 16)] for blk in range(FB))

    def batch_body(kb, _):
        ebase = s * EDGES_PER_TILE + kb * BT
        brow = s * NBATCH + kb
        pltpu.sync_copy(srcb_hbm.at[c * (N_EDGES // BT) + brow], idx_v)
        pltpu.sync_copy(dst2_hbm.at[brow], dst_v)
        pltpu.sync_copy(dst_hbm.at[pl.ds(ebase, BT)], dstf_v)
        pltpu.sync_copy(ea_hbm.at[pl.ds(ebase * EA, BT * EA)], ea_v)

        def sub_body(j, _):
            pltpu.async_copy(p_hbm.at[idx_v.at[j]], rows_v, sem).wait()
            eoff = j * (B * EA)

            def edge_body(i, _):
                ev = ea_v[pl.ds(eoff + i * EA, 16)]
                e0 = ev[0]
                e1 = ev[1]
                e2 = ev[2]
                e3 = ev[3]
                for blk in range(FB):
                    o = blk * 16
                    v = bvecs[blk]
                    v = v + e0 * rows_v[i, pl.ds(o, 16)]
                    v = v + e1 * rows_v[i, pl.ds(HALF + o, 16)]
                    v = v + e2 * rows_v[i, pl.ds(2 * HALF + o, 16)]
                    v = v + e3 * rows_v[i, pl.ds(3 * HALF + o, 16)]
                    msg_v[i, pl.ds(o, 16)] = jnp.maximum(v, 0.0)
                return 0

            lax.fori_loop(0, B, edge_body, 0)
            pltpu.sync_copy(msg_v, acc_sh.at[dst_v.at[j]], add=True)
            return 0

        lax.fori_loop(0, G, sub_body, 0)

        def cnt_body(g, _):
            dv = dstf_v[pl.ds(g * 16, 16)]
            for l in range(16):
                d = dv[l]
                db = (d >> 4) << 4
                oneh = jnp.where(iota == (d & 15), 1.0, 0.0)
                cnt_l[pl.ds(db, 16)] = cnt_l[pl.ds(db, 16)] + oneh
            return 0

        lax.fori_loop(0, BT // 16, cnt_body, 0)
        return 0

    lax.fori_loop(0, NBATCH, batch_body, 0)
    plsc.subcore_barrier()

    for k in range(NFIN):
        nb = s * NODES_PER_TILE + k * B
        pltpu.sync_copy(acc_sh.at[pl.ds(nb, B)], msg_v)
        pltpu.sync_copy(msg_v, acc_hbm.at[pl.ds(c * NPAD + nb, B)])
    pltpu.sync_copy(cnt_l, cnt_hbm.at[pl.ds((c * NS + s) * NPAD, NPAD)])


@functools.partial(
    pl.kernel,
    out_type=[
        jax.ShapeDtypeStruct((NC * NPAD, HALF), jnp.float32),
        jax.ShapeDtypeStruct((NC * NS * NPAD,), jnp.float32),
    ],
    mesh=plsc.VectorSubcoreMesh(core_axis_name="c", subcore_axis_name="s"),
    scratch_types=[
        pltpu.VMEM((G, B), jnp.int32),
        pltpu.VMEM((G, B), jnp.int32),
        pltpu.VMEM((BT,), jnp.int32),
        pltpu.VMEM((BT * EA,), jnp.float32),
        pltpu.VMEM((B, PROJ), jnp.float32),
        pltpu.VMEM((B, HALF), jnp.float32),
        pltpu.VMEM((NPAD,), jnp.float32),
        pltpu.VMEM((HIDDEN_DIM,), jnp.float32),
        pltpu.VMEM((HALF,), jnp.float32),
        pltpu.SemaphoreType.DMA,
        pltpu.VMEM_SHARED((NPAD, HALF), jnp.float32),
    ],
)
def _edge_scatter(p_hbm, srcb_hbm, dst2_hbm, dst_hbm, ea_hbm, b_hbm,
                  acc_hbm, cnt_hbm,
                  idx_v, dst_v, dstf_v, ea_v, rows_v, msg_v, cnt_l,
                  b_v, bh_v, sem, acc_sh):
    _edge_kernel(p_hbm, srcb_hbm, dst2_hbm, dst_hbm, ea_hbm, b_hbm,
                 acc_hbm, cnt_hbm,
                 idx_v, dst_v, dstf_v, ea_v, rows_v, msg_v, cnt_l,
                 b_v, bh_v, sem, acc_sh)


def _mean_kernel(a0_ref, a1_ref, cnt_ref, out_ref):
    tot = jnp.sum(cnt_ref[:NS], axis=0)
    scale = 1.0 / jnp.maximum(tot, 1.0)
    out_ref[:, :HALF] = jnp.maximum(a0_ref[0], 0.0) * scale[:, None]
    out_ref[:, HALF:] = jnp.maximum(a1_ref[0], 0.0) * scale[:, None]


def _mean_relu(acc3, cnt3):
    bn = 2048
    return pl.pallas_call(
        _mean_kernel,
        grid=(NPAD // bn,),
        in_specs=[
            pl.BlockSpec((1, bn, HALF), lambda n: (0, n, 0)),
            pl.BlockSpec((1, bn, HALF), lambda n: (1, n, 0)),
            pl.BlockSpec((NC * NS, bn), lambda n: (0, n)),
        ],
        out_specs=pl.BlockSpec((bn, HIDDEN_DIM), lambda n: (n, 0)),
        out_shape=jax.ShapeDtypeStruct((NPAD, HIDDEN_DIM), jnp.float32),
    )(acc3, acc3, cnt3)


def kernel(node_features, edge_index, edge_attr, W, b):
    a2 = W.reshape(NC, HALF, EDGE_DIM, NODE_DIM).transpose(0, 3, 2, 1)
    a2 = a2.reshape(NC, NODE_DIM, PROJ)
    src = edge_index[0].astype(jnp.int32)
    dst = edge_index[1].astype(jnp.int32)
    srcb = jnp.concatenate([src, src + NPAD]).reshape(-1, G, B)
    dst2 = dst.reshape(-1, G, B)
    ea16 = jnp.pad(edge_attr, ((0, 0), (0, EA - EDGE_DIM))).reshape(-1)

    p = _node_projections(node_features, a2)
    p_flat = p.reshape(NC * NPAD, PROJ)
    acc, cnt = _edge_scatter(p_flat, srcb, dst2, dst, ea16, b)
    acc3 = acc.reshape(NC, NPAD, HALF)
    cnt3 = cnt.reshape(NC * NS, NPAD)
    return _mean_relu(acc3, cnt3)[:N_NODES]


# speedup vs baseline: 1.6762x; 1.2319x over previous
import functools

import jax
import jax.numpy as jnp
from jax import lax
from jax.experimental import pallas as pl
from jax.experimental.pallas import tpu as pltpu
from jax.experimental.pallas import tpu_sc as plsc

N_NODES = 10000
N_EDGES = 160000
NODE_DIM = 128
EDGE_DIM = 4
HIDDEN_DIM = 256

NC = 2
NS = 16
B = 16
EA = 16
EDGES_PER_TILE = N_EDGES // NS
NPAD = 10240
NODES_PER_TILE = NPAD // NS
FINB = 40
NFIN = NODES_PER_TILE // FINB
HALF = HIDDEN_DIM // NC
FB = HALF // 16
PROJ = EDGE_DIM * HALF
G = 25
BT = G * B
NBATCH = EDGES_PER_TILE // BT


def _proj_kernel(nf_ref, a_ref, out_ref):
    out_ref[0] = jnp.dot(nf_ref[...], a_ref[0],
                         preferred_element_type=jnp.float32)


def _node_projections(node_features, a2):
    bn = 1000
    return pl.pallas_call(
        _proj_kernel,
        grid=(NC, N_NODES // bn),
        in_specs=[
            pl.BlockSpec((bn, NODE_DIM), lambda c, n: (n, 0)),
            pl.BlockSpec((1, NODE_DIM, PROJ), lambda c, n: (c, 0, 0)),
        ],
        out_specs=pl.BlockSpec((1, bn, PROJ), lambda c, n: (c, n, 0)),
        out_shape=jax.ShapeDtypeStruct((NC, NPAD, PROJ), jnp.float32),
    )(node_features, a2)


def _edge_kernel(p_hbm, srcb_hbm, dst2_hbm, dst_hbm, ea_hbm, b_hbm,
                 acc_hbm, cnt_hbm,
                 idx_v, dst_v, ea_v, rows0_v, rows1_v, msg_v, fin_v,
                 cnt_l, b_v, bh_v, sem0, sem1, acc_sh):
    c = lax.axis_index("c")
    s = lax.axis_index("s")
    iota = lax.iota(jnp.int32, 16)

    pltpu.sync_copy(b_hbm, b_v)
    for blk in range(FB):
        o = blk * 16
        bh_v[pl.ds(o, 16)] = b_v[pl.ds(c * HALF + o, 16)]

    def zfin(i, _):
        for blk in range(FB):
            fin_v[i, pl.ds(blk * 16, 16)] = jnp.zeros((16,), jnp.float32)
        return 0

    lax.fori_loop(0, FINB, zfin, 0)

    def zcnt(i, _):
        cnt_l[pl.ds(i * 16, 16)] = jnp.zeros((16,), jnp.float32)
        return 0

    lax.fori_loop(0, NPAD // 16, zcnt, 0)

    for k in range(NFIN):
        nb = s * NODES_PER_TILE + k * FINB
        pltpu.sync_copy(fin_v, acc_sh.at[pl.ds(nb, FINB)])
    plsc.subcore_barrier()

    bvecs = tuple(bh_v[pl.ds(blk * 16, 16)] for blk in range(FB))
    bufs = (rows0_v, rows1_v)
    sems = (sem0, sem1)

    def batch_body(kb, _):
        ebase = s * EDGES_PER_TILE + kb * BT
        brow = s * NBATCH + kb
        pltpu.sync_copy(srcb_hbm.at[c * (N_EDGES // BT) + brow], idx_v)
        pltpu.sync_copy(dst2_hbm.at[brow], dst_v)
        pltpu.sync_copy(ea_hbm.at[pl.ds(ebase * EA, BT * EA)], ea_v)

        def compute_chunk(j, rows_v):
            eoff = j * (B * EA)

            def edge_body(i, _):
                ev = ea_v[pl.ds(eoff + i * EA, 16)]
                e0 = ev[0]
                e1 = ev[1]
                e2 = ev[2]
                e3 = ev[3]
                for blk in range(FB):
                    o = blk * 16
                    v = bvecs[blk]
                    v = v + e0 * rows_v[i, pl.ds(o, 16)]
                    v = v + e1 * rows_v[i, pl.ds(HALF + o, 16)]
                    v = v + e2 * rows_v[i, pl.ds(2 * HALF + o, 16)]
                    v = v + e3 * rows_v[i, pl.ds(3 * HALF + o, 16)]
                    msg_v[i, pl.ds(o, 16)] = jnp.maximum(v, 0.0)
                return 0

            lax.fori_loop(0, B, edge_body, 0)
            pltpu.sync_copy(msg_v, acc_sh.at[dst_v.at[j]], add=True)

        pltpu.async_copy(p_hbm.at[idx_v.at[0]], bufs[0], sems[0])

        def pipe_body(j, _):
            par = j & 1
            for p_ in (0, 1):
                @pl.when(par == p_)
                def _():
                    @pl.when(j + 1 < G)
                    def _():
                        pltpu.async_copy(p_hbm.at[idx_v.at[j + 1]],
                                         bufs[1 - p_], sems[1 - p_])
                    pltpu.make_async_copy(p_hbm.at[idx_v.at[j]],
                                          bufs[p_], sems[p_]).wait()
                    compute_chunk(j, bufs[p_])
            return 0

        lax.fori_loop(0, G, pipe_body, 0)

        def cnt_body(g, _):
            dv = dst_v[g, :]
            for l in range(16):
                d = dv[l]
                db = (d >> 4) << 4
                oneh = jnp.where(iota == (d & 15), 1.0, 0.0)
                cnt_l[pl.ds(db, 16)] = cnt_l[pl.ds(db, 16)] + oneh
            return 0

        lax.fori_loop(0, G, cnt_body, 0)
        return 0

    lax.fori_loop(0, NBATCH, batch_body, 0)
    plsc.subcore_barrier()

    for k in range(NFIN):
        nb = s * NODES_PER_TILE + k * FINB
        pltpu.sync_copy(acc_sh.at[pl.ds(nb, FINB)], fin_v)
        pltpu.sync_copy(fin_v, acc_hbm.at[pl.ds(c * NPAD + nb, FINB)])
    pltpu.sync_copy(cnt_l, cnt_hbm.at[pl.ds((c * NS + s) * NPAD, NPAD)])


@functools.partial(
    pl.kernel,
    out_type=[
        jax.ShapeDtypeStruct((NC * NPAD, HALF), jnp.float32),
        jax.ShapeDtypeStruct((NC * NS * NPAD,), jnp.float32),
    ],
    mesh=plsc.VectorSubcoreMesh(core_axis_name="c", subcore_axis_name="s"),
    scratch_types=[
        pltpu.VMEM((G, B), jnp.int32),
        pltpu.VMEM((G, B), jnp.int32),
        pltpu.VMEM((BT * EA,), jnp.float32),
        pltpu.VMEM((B, PROJ), jnp.float32),
        pltpu.VMEM((B, PROJ), jnp.float32),
        pltpu.VMEM((B, HALF), jnp.float32),
        pltpu.VMEM((FINB, HALF), jnp.float32),
        pltpu.VMEM((NPAD,), jnp.float32),
        pltpu.VMEM((HIDDEN_DIM,), jnp.float32),
        pltpu.VMEM((HALF,), jnp.float32),
        pltpu.SemaphoreType.DMA,
        pltpu.SemaphoreType.DMA,
        pltpu.VMEM_SHARED((NPAD, HALF), jnp.float32),
    ],
)
def _edge_scatter(p_hbm, srcb_hbm, dst2_hbm, dst_hbm, ea_hbm, b_hbm,
                  acc_hbm, cnt_hbm,
                  idx_v, dst_v, ea_v, rows0_v, rows1_v, msg_v, fin_v,
                  cnt_l, b_v, bh_v, sem0, sem1, acc_sh):
    _edge_kernel(p_hbm, srcb_hbm, dst2_hbm, dst_hbm, ea_hbm, b_hbm,
                 acc_hbm, cnt_hbm,
                 idx_v, dst_v, ea_v, rows0_v, rows1_v, msg_v, fin_v,
                 cnt_l, b_v, bh_v, sem0, sem1, acc_sh)


def _mean_kernel(a0_ref, a1_ref, cnt_ref, out_ref):
    tot = jnp.sum(cnt_ref[:NS], axis=0)
    scale = 1.0 / jnp.maximum(tot, 1.0)
    out_ref[:, :HALF] = jnp.maximum(a0_ref[0], 0.0) * scale[:, None]
    out_ref[:, HALF:] = jnp.maximum(a1_ref[0], 0.0) * scale[:, None]


def _mean_relu(acc3, cnt3):
    bn = 2048
    return pl.pallas_call(
        _mean_kernel,
        grid=(NPAD // bn,),
        in_specs=[
            pl.BlockSpec((1, bn, HALF), lambda n: (0, n, 0)),
            pl.BlockSpec((1, bn, HALF), lambda n: (1, n, 0)),
            pl.BlockSpec((NC * NS, bn), lambda n: (0, n)),
        ],
        out_specs=pl.BlockSpec((bn, HIDDEN_DIM), lambda n: (n, 0)),
        out_shape=jax.ShapeDtypeStruct((NPAD, HIDDEN_DIM), jnp.float32),
    )(acc3, acc3, cnt3)


def kernel(node_features, edge_index, edge_attr, W, b):
    a2 = W.reshape(NC, HALF, EDGE_DIM, NODE_DIM).transpose(0, 3, 2, 1)
    a2 = a2.reshape(NC, NODE_DIM, PROJ)
    src = edge_index[0].astype(jnp.int32)
    dst = edge_index[1].astype(jnp.int32)
    srcb = jnp.concatenate([src, src + NPAD]).reshape(-1, G, B)
    dst2 = dst.reshape(-1, G, B)
    ea16 = jnp.pad(edge_attr, ((0, 0), (0, EA - EDGE_DIM))).reshape(-1)

    p = _node_projections(node_features, a2)
    p_flat = p.reshape(NC * NPAD, PROJ)
    acc, cnt = _edge_scatter(p_flat, srcb, dst2, dst, ea16, b)
    acc3 = acc.reshape(NC, NPAD, HALF)
    cnt3 = cnt.reshape(NC * NS, NPAD)
    return _mean_relu(acc3, cnt3)[:N_NODES]
